# baseline (speedup 1.0000x reference)
import functools

import jax
import jax.numpy as jnp
from jax import lax
from jax.experimental import pallas as pl
from jax.experimental.pallas import tpu as pltpu
from jax.experimental.pallas import tpu_sc as plsc

BATCH = 4096
SEQ = 200
EMBED = 64
NTOK = BATCH * SEQ

NC = 2
NS = 16
NW = NC * NS

IDX_W = 128
CHUNK = 512
DMAS = CHUNK // IDX_W
TOK_PER_W = NTOK // NW
STEPS = TOK_PER_W // CHUNK
IDX_ROWS_PER_W = TOK_PER_W // IDX_W

@functools.cache
def _build_sc_gather():
    mesh = plsc.VectorSubcoreMesh(core_axis_name="c", subcore_axis_name="s")

    @functools.partial(
        pl.kernel,
        mesh=mesh,
        out_type=jax.ShapeDtypeStruct((NTOK, EMBED), jnp.float32),
        scratch_types=[
            pltpu.VMEM((DMAS, IDX_W), jnp.int32),
            pltpu.VMEM((CHUNK, EMBED), jnp.float32),
            pltpu.SemaphoreType.DMA,
        ],
        compiler_params=pltpu.CompilerParams(use_tc_tiling_on_sc=False),
    )
    def _sc_gather(idx_hbm, table_hbm, out_hbm, idx_v, rows_v, sem):
        wid = lax.axis_index("s") * NC + lax.axis_index("c")
        row_base = wid * IDX_ROWS_PER_W
        out_base = wid * TOK_PER_W

        def step(i, carry):
            pltpu.sync_copy(idx_hbm.at[pl.ds(row_base + i * DMAS, DMAS)], idx_v)
            copies = [
                pltpu.async_copy(
                    table_hbm.at[idx_v.at[j]],
                    rows_v.at[pl.ds(j * IDX_W, IDX_W)],
                    sem,
                )
                for j in range(DMAS)
            ]
            for cp in copies:
                cp.wait()
            pltpu.sync_copy(rows_v, out_hbm.at[pl.ds(out_base + i * CHUNK, CHUNK)])
            return carry

        lax.fori_loop(0, STEPS, step, 0)

    return _sc_gather


def kernel(token_matrix, table):
    idx = token_matrix.astype(jnp.int32).reshape(NTOK // IDX_W, IDX_W)
    out = _build_sc_gather()(idx, table)
    return out.reshape(BATCH, SEQ, EMBED)


# speedup vs baseline: 1.0448x; 1.0448x over previous
import functools

import jax
import jax.numpy as jnp
from jax import lax
from jax.experimental import pallas as pl
from jax.experimental.pallas import tpu as pltpu
from jax.experimental.pallas import tpu_sc as plsc

BATCH = 4096
SEQ = 200
EMBED = 64
NTOK = BATCH * SEQ

NC = 2
NS = 16
NW = NC * NS

IDX_W = 128
CHUNK = 512
DMAS = CHUNK // IDX_W
TOK_PER_W = NTOK // NW
STEPS = TOK_PER_W // CHUNK
IDX_ROWS_PER_W = TOK_PER_W // IDX_W

@functools.cache
def _build_sc_gather():
    mesh = plsc.VectorSubcoreMesh(core_axis_name="c", subcore_axis_name="s")

    @functools.partial(
        pl.kernel,
        mesh=mesh,
        out_type=jax.ShapeDtypeStruct((NTOK, EMBED), jnp.float32),
        scratch_types=[
            pltpu.VMEM((IDX_ROWS_PER_W, IDX_W), jnp.int32),
            pltpu.VMEM((2 * CHUNK, EMBED), jnp.float32),
            pltpu.SemaphoreType.DMA,
            pltpu.SemaphoreType.DMA,
            pltpu.SemaphoreType.DMA,
            pltpu.SemaphoreType.DMA,
        ],
        compiler_params=pltpu.CompilerParams(use_tc_tiling_on_sc=False),
    )
    def _sc_gather(idx_hbm, table_hbm, out_hbm, idx_v, rows_v, sg0, sg1, so0, so1):
        wid = lax.axis_index("s") * NC + lax.axis_index("c")
        row_base = wid * IDX_ROWS_PER_W
        out_base = wid * TOK_PER_W
        sg = (sg0, sg1)
        so = (so0, so1)

        pltpu.sync_copy(idx_hbm.at[pl.ds(row_base, IDX_ROWS_PER_W)], idx_v)

        def gathers(s, b):
            cps = [
                pltpu.async_copy(
                    table_hbm.at[idx_v.at[s * DMAS + j]],
                    rows_v.at[pl.ds(b * CHUNK + j * IDX_W, IDX_W)],
                    sg[b],
                )
                for j in range(DMAS)
            ]
            for cp in cps:
                cp.wait()

        def start_out(s, b):
            pltpu.async_copy(
                rows_v.at[pl.ds(b * CHUNK, CHUNK)],
                out_hbm.at[pl.ds(out_base + s * CHUNK, CHUNK)],
                so[b],
            )

        def wait_out(b):
            pltpu.make_async_copy(
                rows_v.at[pl.ds(b * CHUNK, CHUNK)],
                out_hbm.at[pl.ds(out_base, CHUNK)],
                so[b],
            ).wait()

        for b in range(2):
            gathers(b, b)
            start_out(b, b)

        def body(k, carry):
            for b in range(2):
                s = 2 * k + b
                wait_out(b)
                gathers(s, b)
                start_out(s, b)
            return carry

        lax.fori_loop(1, STEPS // 2, body, 0)
        wait_out(0)
        wait_out(1)

    return _sc_gather


def kernel(token_matrix, table):
    idx = token_matrix.astype(jnp.int32).reshape(NTOK // IDX_W, IDX_W)
    out = _build_sc_gather()(idx, table)
    return out.reshape(BATCH, SEQ, EMBED)
